# baseline (speedup 1.0000x reference)
import functools

import jax
import jax.numpy as jnp
from jax import lax
from jax.experimental import pallas as pl
from jax.experimental.pallas import tpu as pltpu
from jax.experimental.pallas import tpu_sc as plsc

L = 16
NUM_WORKERS = 32
NBUF = 4
CH_ROWS = 16


def _row_argmax(buf, roff, vecs):
    lane = lax.iota(jnp.int32, L)
    span = vecs // 4

    ms = []
    cs = []
    for k in range(4):
        ms.append(buf[pl.ds(roff + (k * span) * L, L)])
        cs.append(jnp.full((L,), k * span, jnp.int32))

    def vec_body(j, carry):
        out = []
        for k in range(4):
            m, c = carry[2 * k], carry[2 * k + 1]
            jj = k * span + j
            v = buf[pl.ds(roff + jj * L, L)]
            gt = v > m
            m = jnp.where(gt, v, m)
            c = jnp.where(gt, jj, c)
            out += [m, c]
        return tuple(out)

    init = (ms[0], cs[0], ms[1], cs[1], ms[2], cs[2], ms[3], cs[3])
    carry = lax.fori_loop(1, span, vec_body, init, unroll=True)

    m, c = carry[0], carry[1]
    for k in range(1, 4):
        mk, ck = carry[2 * k], carry[2 * k + 1]
        gt = mk > m
        m = jnp.where(gt, mk, m)
        c = jnp.where(gt, ck, c)

    gmax = jnp.max(m)
    idx = c * L + lane
    cand = jnp.where(m == gmax, idx, vecs * L)
    return jnp.min(cand)


@functools.lru_cache(maxsize=None)
def _build(rows, vocab):
    assert rows % (NUM_WORKERS * NBUF * CH_ROWS) == 0
    assert vocab % (4 * L) == 0
    rows_per_w = rows // NUM_WORKERS
    n_chunks = rows_per_w // CH_ROWS
    chunk = CH_ROWS * vocab
    vecs = vocab // L

    mesh = plsc.VectorSubcoreMesh(core_axis_name="c", subcore_axis_name="s")

    @functools.partial(
        pl.kernel,
        out_type=jax.ShapeDtypeStruct((rows,), jnp.int32),
        mesh=mesh,
        compiler_params=pltpu.CompilerParams(needs_layout_passes=False),
        scratch_types=(
            [pltpu.VMEM((chunk,), jnp.float32) for _ in range(NBUF)]
            + [pltpu.VMEM((rows_per_w,), jnp.int32)]
            + [pltpu.SemaphoreType.DMA for _ in range(NBUF)]
        ),
    )
    def k(x_hbm, out_hbm, b0, b1, b2, b3, out_v, s0, s1, s2, s3):
        bufs = (b0, b1, b2, b3)
        sems = (s0, s1, s2, s3)
        wid = lax.axis_index("s") * 2 + lax.axis_index("c")
        base = wid * rows_per_w * vocab

        def start(g, b):
            pltpu.async_copy(
                x_hbm.at[pl.ds(base + g * chunk, chunk)], bufs[b], sems[b])

        def wait(b):
            pltpu.make_async_copy(
                x_hbm.at[pl.ds(0, chunk)], bufs[b], sems[b]).wait()

        def process(g, b):
            buf = bufs[b]

            def half(h, _):
                def row_body(r16, resvec):
                    res = _row_argmax(buf, (h * L + r16) * vocab, vecs)
                    return jnp.where(lax.iota(jnp.int32, L) == r16, res, resvec)

                resvec = lax.fori_loop(0, L, row_body, jnp.zeros((L,), jnp.int32))
                out_v[pl.ds(g * CH_ROWS + h * L, L)] = resvec
                return 0

            lax.fori_loop(0, CH_ROWS // L, half, 0)

        for b in range(NBUF):
            start(b, b)

        def ring(i, _):
            for b in range(NBUF):
                g = i * NBUF + b
                wait(b)
                process(g, b)
                start(g + NBUF, b)
            return 0

        lax.fori_loop(0, n_chunks // NBUF - 1, ring, 0)
        for b in range(NBUF):
            g = n_chunks - NBUF + b
            wait(b)
            process(g, b)

        pltpu.sync_copy(out_v, out_hbm.at[pl.ds(wid * rows_per_w, rows_per_w)])

    return k


def kernel(log_probs):
    b, t, v = log_probs.shape
    out = _build(b * t, v)(log_probs.reshape(-1))
    return out.reshape(b, t)


# speedup vs baseline: 2.2889x; 2.2889x over previous
import functools

import jax
import jax.numpy as jnp
from jax import lax
from jax.experimental import pallas as pl
from jax.experimental.pallas import tpu as pltpu
from jax.experimental.pallas import tpu_sc as plsc

L = 16
NUM_WORKERS = 32
NBUF = 4
CH_ROWS = 16


def _row_argmax(buf, row, vecs):
    lane = lax.iota(jnp.int32, L)
    span = vecs // 4

    ms = []
    cs = []
    for k in range(4):
        ms.append(buf[row, pl.ds((k * span) * L, L)])
        cs.append(jnp.full((L,), k * span, jnp.int32))

    def vec_body(j, carry):
        out = []
        for k in range(4):
            m, c = carry[2 * k], carry[2 * k + 1]
            jj = k * span + j
            v = buf[row, pl.ds(jj * L, L)]
            gt = v > m
            m = jnp.where(gt, v, m)
            c = jnp.where(gt, jj, c)
            out += [m, c]
        return tuple(out)

    init = (ms[0], cs[0], ms[1], cs[1], ms[2], cs[2], ms[3], cs[3])
    carry = lax.fori_loop(1, span, vec_body, init, unroll=True)

    m, c = carry[0], carry[1]
    for k in range(1, 4):
        mk, ck = carry[2 * k], carry[2 * k + 1]
        gt = mk > m
        m = jnp.where(gt, mk, m)
        c = jnp.where(gt, ck, c)

    gmax = jnp.max(m)
    idx = c * L + lane
    cand = jnp.where(m == gmax, idx, vecs * L)
    return jnp.min(cand)


@functools.lru_cache(maxsize=None)
def _build(rows, vocab):
    assert rows % (NUM_WORKERS * NBUF * CH_ROWS) == 0
    assert vocab % (4 * L) == 0
    rows_per_w = rows // NUM_WORKERS
    n_chunks = rows_per_w // CH_ROWS
    vecs = vocab // L

    mesh = plsc.VectorSubcoreMesh(core_axis_name="c", subcore_axis_name="s")

    @functools.partial(
        pl.kernel,
        out_type=jax.ShapeDtypeStruct((rows,), jnp.int32),
        mesh=mesh,
        compiler_params=pltpu.CompilerParams(needs_layout_passes=False),
        scratch_types=(
            [pltpu.VMEM((CH_ROWS, vocab), jnp.float32) for _ in range(NBUF)]
            + [pltpu.VMEM((rows_per_w,), jnp.int32)]
            + [pltpu.SemaphoreType.DMA for _ in range(NBUF)]
        ),
    )
    def k(x_hbm, out_hbm, b0, b1, b2, b3, out_v, s0, s1, s2, s3):
        bufs = (b0, b1, b2, b3)
        sems = (s0, s1, s2, s3)
        wid = lax.axis_index("s") * 2 + lax.axis_index("c")
        row0 = wid * rows_per_w

        def start(g, b):
            pltpu.async_copy(
                x_hbm.at[pl.ds(row0 + g * CH_ROWS, CH_ROWS)], bufs[b], sems[b])

        def wait(b):
            pltpu.make_async_copy(
                x_hbm.at[pl.ds(0, CH_ROWS)], bufs[b], sems[b]).wait()

        def process(g, b):
            buf = bufs[b]

            def half(h, _):
                def row_body(r16, resvec):
                    res = _row_argmax(buf, h * L + r16, vecs)
                    return jnp.where(lax.iota(jnp.int32, L) == r16, res, resvec)

                resvec = lax.fori_loop(0, L, row_body, jnp.zeros((L,), jnp.int32))
                out_v[pl.ds(g * CH_ROWS + h * L, L)] = resvec
                return 0

            lax.fori_loop(0, CH_ROWS // L, half, 0)

        for b in range(NBUF):
            start(b, b)

        def ring(i, _):
            for b in range(NBUF):
                g = i * NBUF + b
                wait(b)
                process(g, b)
                start(g + NBUF, b)
            return 0

        lax.fori_loop(0, n_chunks // NBUF - 1, ring, 0)
        for b in range(NBUF):
            g = n_chunks - NBUF + b
            wait(b)
            process(g, b)

        pltpu.sync_copy(out_v, out_hbm.at[pl.ds(wid * rows_per_w, rows_per_w)])

    return k


def kernel(log_probs):
    b, t, v = log_probs.shape
    out = _build(b * t, v)(log_probs.reshape(b * t, v))
    return out.reshape(b, t)


# speedup vs baseline: 2.5606x; 1.1187x over previous
import functools

import jax
import jax.numpy as jnp
from jax import lax
from jax.experimental import pallas as pl
from jax.experimental.pallas import tpu as pltpu
from jax.experimental.pallas import tpu_sc as plsc

L = 16
NUM_WORKERS = 32
NBUF = 4
CH_ROWS = 16


def _row_maxc(buf, row, vecs):
    span = vecs // 4

    ms = []
    cs = []
    for k in range(4):
        ms.append(buf[row, pl.ds((k * span) * L, L)])
        cs.append(jnp.full((L,), k * span, jnp.int32))

    def vec_body(j, carry):
        out = []
        for k in range(4):
            m, c = carry[2 * k], carry[2 * k + 1]
            jj = k * span + j
            v = buf[row, pl.ds(jj * L, L)]
            gt = v > m
            m = jnp.where(gt, v, m)
            c = jnp.where(gt, jj, c)
            out += [m, c]
        return tuple(out)

    init = (ms[0], cs[0], ms[1], cs[1], ms[2], cs[2], ms[3], cs[3])
    carry = lax.fori_loop(1, span, vec_body, init, unroll=True)

    m, c = carry[0], carry[1]
    for k in range(1, 4):
        mk, ck = carry[2 * k], carry[2 * k + 1]
        gt = mk > m
        m = jnp.where(gt, mk, m)
        c = jnp.where(gt, ck, c)
    return m, c


@functools.lru_cache(maxsize=None)
def _build(rows, vocab):
    assert rows % (NUM_WORKERS * NBUF * CH_ROWS) == 0
    assert vocab % (4 * L) == 0
    rows_per_w = rows // NUM_WORKERS
    n_chunks = rows_per_w // CH_ROWS
    vecs = vocab // L

    mesh = plsc.VectorSubcoreMesh(core_axis_name="c", subcore_axis_name="s")

    @functools.partial(
        pl.kernel,
        out_type=jax.ShapeDtypeStruct((rows,), jnp.int32),
        mesh=mesh,
        compiler_params=pltpu.CompilerParams(needs_layout_passes=False),
        scratch_types=(
            [pltpu.VMEM((CH_ROWS, vocab), jnp.float32) for _ in range(NBUF)]
            + [pltpu.VMEM((rows_per_w,), jnp.int32),
               pltpu.VMEM((CH_ROWS * L,), jnp.float32),
               pltpu.VMEM((CH_ROWS * L,), jnp.int32)]
            + [pltpu.SemaphoreType.DMA for _ in range(NBUF)]
        ),
    )
    def k(x_hbm, out_hbm, b0, b1, b2, b3, out_v, mbuf, cbuf, s0, s1, s2, s3):
        bufs = (b0, b1, b2, b3)
        sems = (s0, s1, s2, s3)
        wid = lax.axis_index("s") * 2 + lax.axis_index("c")
        row0 = wid * rows_per_w

        def start(g, b):
            pltpu.async_copy(
                x_hbm.at[pl.ds(row0 + g * CH_ROWS, CH_ROWS)], bufs[b], sems[b])

        def wait(b):
            pltpu.make_async_copy(
                x_hbm.at[pl.ds(0, CH_ROWS)], bufs[b], sems[b]).wait()

        def process(g, b):
            buf = bufs[b]

            def row_body(r, _):
                m, c = _row_maxc(buf, r, vecs)
                mbuf[pl.ds(r * L, L)] = m
                cbuf[pl.ds(r * L, L)] = c
                return 0

            lax.fori_loop(0, CH_ROWS, row_body, 0)

            lane = lax.iota(jnp.int32, L)
            stride = lane * L
            best = plsc.load_gather(mbuf, [stride])
            besti = plsc.load_gather(cbuf, [stride]) * L
            for l in range(1, L):
                v = plsc.load_gather(mbuf, [stride + l])
                vi = plsc.load_gather(cbuf, [stride + l]) * L + l
                gt = v > best
                eq = v == best
                lt = vi < besti
                upd = gt | (eq & lt)
                best = jnp.where(upd, v, best)
                besti = jnp.where(upd, vi, besti)
            out_v[pl.ds(g * CH_ROWS, L)] = besti

        for b in range(NBUF):
            start(b, b)

        def ring(i, _):
            for b in range(NBUF):
                g = i * NBUF + b
                wait(b)
                process(g, b)
                start(g + NBUF, b)
            return 0

        lax.fori_loop(0, n_chunks // NBUF - 1, ring, 0)
        for b in range(NBUF):
            g = n_chunks - NBUF + b
            wait(b)
            process(g, b)

        pltpu.sync_copy(out_v, out_hbm.at[pl.ds(wid * rows_per_w, rows_per_w)])

    return k


def kernel(log_probs):
    b, t, v = log_probs.shape
    out = _build(b * t, v)(log_probs.reshape(b * t, v))
    return out.reshape(b, t)


# speedup vs baseline: 2.7941x; 1.0912x over previous
import functools

import jax
import jax.numpy as jnp
from jax import lax
from jax.experimental import pallas as pl
from jax.experimental.pallas import tpu as pltpu
from jax.experimental.pallas import tpu_sc as plsc

L = 16
NUM_WORKERS = 32
NBUF = 4
CH_ROWS = 16


def _row_maxc(buf, row, vecs):
    span = vecs // 4

    ms = []
    cs = []
    for k in range(4):
        ms.append(buf[row, pl.ds((k * span) * L, L)])
        cs.append(jnp.full((L,), k * span, jnp.int32))

    def vec_body(j, carry):
        out = []
        for k in range(4):
            m, c = carry[2 * k], carry[2 * k + 1]
            jj = k * span + j
            v = buf[row, pl.ds(jj * L, L)]
            gt = v > m
            m = jnp.where(gt, v, m)
            c = jnp.where(gt, jj, c)
            out += [m, c]
        return tuple(out)

    init = (ms[0], cs[0], ms[1], cs[1], ms[2], cs[2], ms[3], cs[3])
    carry = lax.fori_loop(1, span, vec_body, init, unroll=True)

    m, c = carry[0], carry[1]
    for k in range(1, 4):
        mk, ck = carry[2 * k], carry[2 * k + 1]
        gt = mk > m
        m = jnp.where(gt, mk, m)
        c = jnp.where(gt, ck, c)
    return m, c


@functools.lru_cache(maxsize=None)
def _build(rows, vocab):
    assert rows % (NUM_WORKERS * NBUF * CH_ROWS) == 0
    assert vocab % (4 * L) == 0
    rows_per_w = rows // NUM_WORKERS
    n_chunks = rows_per_w // CH_ROWS
    vecs = vocab // L

    mesh = plsc.VectorSubcoreMesh(core_axis_name="c", subcore_axis_name="s")

    @functools.partial(
        pl.kernel,
        out_type=jax.ShapeDtypeStruct((rows,), jnp.int32),
        mesh=mesh,
        compiler_params=pltpu.CompilerParams(needs_layout_passes=False),
        scratch_types=(
            [pltpu.VMEM((CH_ROWS, vocab), jnp.float32) for _ in range(NBUF)]
            + [pltpu.VMEM((rows_per_w,), jnp.int32),
               pltpu.VMEM((CH_ROWS * L,), jnp.float32),
               pltpu.VMEM((CH_ROWS * L,), jnp.int32)]
            + [pltpu.SemaphoreType.DMA for _ in range(NBUF)]
        ),
    )
    def k(x_hbm, out_hbm, b0, b1, b2, b3, out_v, mbuf, cbuf, s0, s1, s2, s3):
        bufs = (b0, b1, b2, b3)
        sems = (s0, s1, s2, s3)
        wid = lax.axis_index("s") * 2 + lax.axis_index("c")
        row0 = wid * rows_per_w

        def start(g, b):
            pltpu.async_copy(
                x_hbm.at[pl.ds(row0 + g * CH_ROWS, CH_ROWS)], bufs[b], sems[b])

        def wait(b):
            pltpu.make_async_copy(
                x_hbm.at[pl.ds(0, CH_ROWS)], bufs[b], sems[b]).wait()

        def process(g, b):
            buf = bufs[b]

            @plsc.parallel_loop(0, CH_ROWS, unroll=2)
            def row_body(r):
                m, c = _row_maxc(buf, r, vecs)
                mbuf[pl.ds(r * L, L)] = m
                cbuf[pl.ds(r * L, L)] = c

            lane = lax.iota(jnp.int32, L)
            stride = lane * L
            best = plsc.load_gather(mbuf, [stride])
            besti = plsc.load_gather(cbuf, [stride]) * L
            for l in range(1, L):
                v = plsc.load_gather(mbuf, [stride + l])
                vi = plsc.load_gather(cbuf, [stride + l]) * L + l
                gt = v > best
                eq = v == best
                lt = vi < besti
                upd = gt | (eq & lt)
                best = jnp.where(upd, v, best)
                besti = jnp.where(upd, vi, besti)
            out_v[pl.ds(g * CH_ROWS, L)] = besti

        for b in range(NBUF):
            start(b, b)

        def ring(i, _):
            for b in range(NBUF):
                g = i * NBUF + b
                wait(b)
                process(g, b)
                start(g + NBUF, b)
            return 0

        lax.fori_loop(0, n_chunks // NBUF - 1, ring, 0)
        for b in range(NBUF):
            g = n_chunks - NBUF + b
            wait(b)
            process(g, b)

        pltpu.sync_copy(out_v, out_hbm.at[pl.ds(wid * rows_per_w, rows_per_w)])

    return k


def kernel(log_probs):
    b, t, v = log_probs.shape
    out = _build(b * t, v)(log_probs.reshape(b * t, v))
    return out.reshape(b, t)
